# baseline (speedup 1.0000x reference)
import functools

import jax
import jax.numpy as jnp
from jax import lax
from jax.experimental import pallas as pl
from jax.experimental.pallas import tpu as pltpu
from jax.experimental.pallas import tpu_sc as plsc

NUM_QUESTS = 512
NUM_PATHS = 4
LINK_DIM = 256
NUM_LINKS = 10000
MAX_LEN = 32
B = NUM_QUESTS * NUM_PATHS

NBLK = 16
BLKB = B // NBLK
HALF_T = MAX_LEN // 2
CHUNKBUF = BLKB * MAX_LEN + 32


def _proj_body(x_ref, w_ref, p_ref, ss_ref):
    i = pl.program_id(0)
    x = x_ref[...]
    p_ref[...] = jnp.dot(x, w_ref[...], preferred_element_type=jnp.float32)
    blk = jnp.sum(x * x)

    @pl.when(i == 0)
    def _():
        ss_ref[0, 0] = blk

    @pl.when(i > 0)
    def _():
        ss_ref[0, 0] += blk


def _project(inputs, w_pad):
    n_blk = 10
    rows = NUM_LINKS // n_blk
    return pl.pallas_call(
        _proj_body,
        grid=(n_blk,),
        in_specs=[
            pl.BlockSpec((rows, LINK_DIM), lambda i: (i, 0)),
            pl.BlockSpec((LINK_DIM, 4), lambda i: (0, 0)),
        ],
        out_specs=[
            pl.BlockSpec((rows, 4), lambda i: (i, 0)),
            pl.BlockSpec((1, 1), lambda i: (0, 0), memory_space=pltpu.SMEM),
        ],
        out_shape=[
            jax.ShapeDtypeStruct((NUM_LINKS, 4), jnp.float32),
            jax.ShapeDtypeStruct((1, 1), jnp.float32),
        ],
    )(inputs, w_pad)


def _make_scatter(padlen, total):
    mesh = plsc.VectorSubcoreMesh(core_axis_name="c", subcore_axis_name="s")
    cap = padlen - CHUNKBUF

    @functools.partial(
        pl.kernel,
        mesh=mesh,
        compiler_params=pltpu.CompilerParams(needs_layout_passes=False),
        out_type=[
            jax.ShapeDtypeStruct((MAX_LEN, B), jnp.float32),
            jax.ShapeDtypeStruct((MAX_LEN, B), jnp.float32),
            jax.ShapeDtypeStruct((MAX_LEN, B), jnp.float32),
            jax.ShapeDtypeStruct((2 * B,), jnp.int32),
        ],
        scratch_types=[
            pltpu.VMEM((padlen,), jnp.int32),
            pltpu.VMEM((4 * NUM_LINKS,), jnp.float32),
            pltpu.VMEM((CHUNKBUF,), jnp.int32),
            pltpu.VMEM((CHUNKBUF,), jnp.int32),
            pltpu.VMEM((HALF_T, BLKB), jnp.float32),
            pltpu.VMEM((HALF_T, BLKB), jnp.float32),
            pltpu.VMEM((HALF_T, BLKB), jnp.float32),
            pltpu.VMEM((BLKB,), jnp.int32),
        ],
    )
    def scatter_kernel(p_hbm, idx_hbm, seq_hbm, path_hbm,
                       oz, orr, oh, olens,
                       idx_v, p_v, path_v, seq_v, sz, sr, sh, slens):
        blk = lax.axis_index("s")
        par = lax.axis_index("c")
        b0 = blk * BLKB
        t0 = par * HALF_T

        pltpu.sync_copy(idx_hbm, idx_v)
        pltpu.sync_copy(p_hbm, p_v)

        n_search = max(1, (total + 1).bit_length())

        def lower_bound(target):
            def body(_, st):
                lo, hi = st
                mid = (lo + hi) // 2
                probe = jnp.full((16,), mid, jnp.int32)
                v = jnp.max(plsc.load_gather(idx_v, [probe]))
                go_right = v < target
                return (jnp.where(go_right, mid + 1, lo),
                        jnp.where(go_right, hi, mid))

            lo, _ = lax.fori_loop(0, n_search, body,
                                  (jnp.int32(0), jnp.int32(total)))
            return lo

        lo_w = lower_bound(b0)
        hi_w = lower_bound(b0 + BLKB)
        a0 = (lo_w // 16) * 16
        cs = jnp.minimum(a0, jnp.int32(cap))

        pltpu.sync_copy(path_hbm.at[pl.ds(cs, CHUNKBUF)], path_v)
        pltpu.sync_copy(seq_hbm.at[pl.ds(cs, CHUNKBUF)], seq_v)

        zeros16 = jnp.zeros((16,), jnp.int32)
        for j in range(BLKB // 16):
            slens[pl.ds(j * 16, 16)] = zeros16

        lane = lax.iota(jnp.int32, 16)
        ones16 = jnp.ones((16,), jnp.int32)
        niter = (hi_w - a0 + 15) // 16
        rel = a0 - cs

        def body(i, carry):
            off = i * 16
            k = a0 + off
            idxv = idx_v[pl.ds(k, 16)]
            pathv = path_v[pl.ds(rel + off, 16)]
            seqv = seq_v[pl.ds(rel + off, 16)]
            kk = k + lane
            m = ((kk >= lo_w) & (kk < hi_w)
                 & (seqv >= t0) & (seqv < t0 + HALF_T))
            bl = jnp.where(m, idxv - b0, 0)
            tt = jnp.where(m, seqv - t0, 0)
            pb = jnp.where(m, pathv, 0) * 4
            vz = plsc.load_gather(p_v, [pb])
            vr = plsc.load_gather(p_v, [pb + 1])
            vh = plsc.load_gather(p_v, [pb + 2])
            plsc.store_scatter(sz, [tt, bl], vz, mask=m)
            plsc.store_scatter(sr, [tt, bl], vr, mask=m)
            plsc.store_scatter(sh, [tt, bl], vh, mask=m)
            plsc.addupdate_scatter(slens, [bl], ones16, mask=m)
            return carry

        lax.fori_loop(0, niter, body, jnp.int32(0))

        pltpu.sync_copy(slens, olens.at[pl.ds(par * B + b0, BLKB)])
        pltpu.sync_copy(sz, oz.at[pl.ds(t0, HALF_T), pl.ds(b0, BLKB)])
        pltpu.sync_copy(sr, orr.at[pl.ds(t0, HALF_T), pl.ds(b0, BLKB)])
        pltpu.sync_copy(sh, oh.at[pl.ds(t0, HALF_T), pl.ds(b0, BLKB)])

    return scatter_kernel


def _gru_body(ss_ref, rk_ref, b_ref, mz_ref, mr_ref, mh_ref, lens_ref, h0_ref,
              out_ref):
    s = lax.rsqrt(jnp.maximum(ss_ref[0, 0], 1e-12))
    rk0 = rk_ref[0, 0]
    rk1 = rk_ref[0, 1]
    rk2 = rk_ref[0, 2]
    b00 = b_ref[0, 0]
    b01 = b_ref[0, 1]
    b02 = b_ref[0, 2]
    b10 = b_ref[1, 0]
    b11 = b_ref[1, 1]
    b12 = b_ref[1, 2]
    h = h0_ref[...]
    lens = lens_ref[0] + lens_ref[1]
    for t in range(MAX_LEN):
        xz = mz_ref[t] * s + b00
        xr = mr_ref[t] * s + b01
        xh = mh_ref[t] * s + b02
        z = jax.nn.sigmoid(xz + h * rk0 + b10)
        r = jax.nn.sigmoid(xr + h * rk1 + b11)
        hh = jnp.tanh(xh + r * (h * rk2 + b12))
        h = jnp.where(t < lens, z * h + (1.0 - z) * hh, h)
    out_ref[...] = h


def _gru_scan(ss, rk, bias, mz, mr, mh, lens, h0):
    smem = pl.BlockSpec(memory_space=pltpu.SMEM)
    vmem = pl.BlockSpec(memory_space=pltpu.VMEM)
    return pl.pallas_call(
        _gru_body,
        in_specs=[smem, smem, smem, vmem, vmem, vmem, vmem, vmem],
        out_specs=pl.BlockSpec(memory_space=pltpu.VMEM),
        out_shape=jax.ShapeDtypeStruct((16, 128), jnp.float32),
    )(ss, rk, bias, mz, mr, mh, lens, h0)


def kernel(inputs, paths, index, sequences, features, flow_size,
           kernel, recurrent_kernel, bias):
    del features
    w_pad = jnp.pad(kernel, ((0, 0), (0, 1)))
    p, ss = _project(inputs, w_pad)

    total = paths.shape[0]
    padlen = total + (-total) % 16 + 32
    pad = padlen - total
    idx_p = jnp.pad(index, (0, pad), constant_values=B)
    seq_p = jnp.pad(sequences, (0, pad))
    path_p = jnp.pad(paths, (0, pad))

    mz, mr, mh, lens = _make_scatter(padlen, total)(
        p.reshape(-1), idx_p, seq_p, path_p)

    out = _gru_scan(ss, recurrent_kernel, bias,
                    mz.reshape(MAX_LEN, 16, 128),
                    mr.reshape(MAX_LEN, 16, 128),
                    mh.reshape(MAX_LEN, 16, 128),
                    lens.reshape(2, 16, 128),
                    flow_size.reshape(16, 128))
    return out.reshape(NUM_QUESTS, NUM_PATHS)


# speedup vs baseline: 1.1787x; 1.1787x over previous
import functools

import jax
import jax.numpy as jnp
from jax import lax
from jax.experimental import pallas as pl
from jax.experimental.pallas import tpu as pltpu
from jax.experimental.pallas import tpu_sc as plsc

NUM_QUESTS = 512
NUM_PATHS = 4
LINK_DIM = 256
NUM_LINKS = 10000
MAX_LEN = 32
B = NUM_QUESTS * NUM_PATHS

NBLK = 16
BLKB = B // NBLK
HALF_T = MAX_LEN // 2
CHUNKBUF = BLKB * MAX_LEN + 32


PROJ_ROWS = 1024
PSTRIDE = 10016


def _proj_body(x_ref, w_ref, p_ref, ss_ref):
    i = pl.program_id(0)
    x = x_ref[...]
    p_ref[...] = lax.dot_general(w_ref[...], x, (((1,), (1,)), ((), ())),
                                 preferred_element_type=jnp.float32)
    row = lax.broadcasted_iota(jnp.int32, (PROJ_ROWS, 1), 0) + i * PROJ_ROWS
    xm = jnp.where(row < NUM_LINKS, x, 0.0)
    blk = jnp.sum(xm * xm)

    @pl.when(i == 0)
    def _():
        ss_ref[0, 0] = blk

    @pl.when(i > 0)
    def _():
        ss_ref[0, 0] += blk


def _project(inputs, w_t):
    n_blk = pl.cdiv(NUM_LINKS, PROJ_ROWS)
    return pl.pallas_call(
        _proj_body,
        grid=(n_blk,),
        in_specs=[
            pl.BlockSpec((PROJ_ROWS, LINK_DIM), lambda i: (i, 0)),
            pl.BlockSpec((4, LINK_DIM), lambda i: (0, 0)),
        ],
        out_specs=[
            pl.BlockSpec((4, PROJ_ROWS), lambda i: (0, i)),
            pl.BlockSpec((1, 1), lambda i: (0, 0), memory_space=pltpu.SMEM),
        ],
        out_shape=[
            jax.ShapeDtypeStruct((4, PSTRIDE), jnp.float32),
            jax.ShapeDtypeStruct((1, 1), jnp.float32),
        ],
    )(inputs, w_t)


def _make_scatter(padlen, total):
    mesh = plsc.VectorSubcoreMesh(core_axis_name="c", subcore_axis_name="s")
    cap = padlen - CHUNKBUF

    @functools.partial(
        pl.kernel,
        mesh=mesh,
        compiler_params=pltpu.CompilerParams(needs_layout_passes=False),
        out_type=[
            jax.ShapeDtypeStruct((NBLK, MAX_LEN, BLKB), jnp.float32),
            jax.ShapeDtypeStruct((NBLK, MAX_LEN, BLKB), jnp.float32),
            jax.ShapeDtypeStruct((NBLK, MAX_LEN, BLKB), jnp.float32),
            jax.ShapeDtypeStruct((2 * B,), jnp.int32),
        ],
        scratch_types=[
            pltpu.VMEM((padlen,), jnp.int32),
            pltpu.VMEM((4 * PSTRIDE,), jnp.float32),
            pltpu.VMEM((CHUNKBUF,), jnp.int32),
            pltpu.VMEM((CHUNKBUF,), jnp.int32),
            pltpu.VMEM((HALF_T, BLKB), jnp.float32),
            pltpu.VMEM((HALF_T, BLKB), jnp.float32),
            pltpu.VMEM((HALF_T, BLKB), jnp.float32),
            pltpu.VMEM((BLKB,), jnp.int32),
        ],
    )
    def scatter_kernel(p_hbm, idx_hbm, seq_hbm, path_hbm,
                       oz, orr, oh, olens,
                       idx_v, p_v, path_v, seq_v, sz, sr, sh, slens):
        blk = lax.axis_index("s")
        par = lax.axis_index("c")
        b0 = blk * BLKB
        t0 = par * HALF_T

        pltpu.sync_copy(idx_hbm, idx_v)
        pltpu.sync_copy(p_hbm, p_v)

        n_search = max(1, (total + 1).bit_length())

        def lower_bound(target):
            def body(_, st):
                lo, hi = st
                mid = (lo + hi) // 2
                probe = jnp.full((16,), mid, jnp.int32)
                v = jnp.max(plsc.load_gather(idx_v, [probe]))
                go_right = v < target
                return (jnp.where(go_right, mid + 1, lo),
                        jnp.where(go_right, hi, mid))

            lo, _ = lax.fori_loop(0, n_search, body,
                                  (jnp.int32(0), jnp.int32(total)))
            return lo

        lo_w = lower_bound(b0)
        hi_w = lower_bound(b0 + BLKB)
        a0 = (lo_w // 16) * 16
        cs = jnp.minimum(a0, jnp.int32(cap))

        pltpu.sync_copy(path_hbm.at[pl.ds(cs, CHUNKBUF)], path_v)
        pltpu.sync_copy(seq_hbm.at[pl.ds(cs, CHUNKBUF)], seq_v)

        zeros16 = jnp.zeros((16,), jnp.int32)
        for j in range(BLKB // 16):
            slens[pl.ds(j * 16, 16)] = zeros16

        lane = lax.iota(jnp.int32, 16)
        ones16 = jnp.ones((16,), jnp.int32)
        niter = (hi_w - a0 + 15) // 16
        rel = a0 - cs

        def body(i, carry):
            off = i * 16
            k = a0 + off
            idxv = idx_v[pl.ds(k, 16)]
            pathv = path_v[pl.ds(rel + off, 16)]
            seqv = seq_v[pl.ds(rel + off, 16)]
            kk = k + lane
            m = ((kk >= lo_w) & (kk < hi_w)
                 & (seqv >= t0) & (seqv < t0 + HALF_T))
            bl = jnp.where(m, idxv - b0, 0)
            tt = jnp.where(m, seqv - t0, 0)
            pb = jnp.where(m, pathv, 0)
            vz = plsc.load_gather(p_v, [pb])
            vr = plsc.load_gather(p_v, [pb + PSTRIDE])
            vh = plsc.load_gather(p_v, [pb + 2 * PSTRIDE])
            plsc.store_scatter(sz, [tt, bl], vz, mask=m)
            plsc.store_scatter(sr, [tt, bl], vr, mask=m)
            plsc.store_scatter(sh, [tt, bl], vh, mask=m)
            plsc.addupdate_scatter(slens, [bl], ones16, mask=m)
            return carry

        lax.fori_loop(0, niter, body, jnp.int32(0))

        pltpu.sync_copy(slens, olens.at[pl.ds(par * B + b0, BLKB)])
        pltpu.sync_copy(sz, oz.at[blk, pl.ds(t0, HALF_T), :])
        pltpu.sync_copy(sr, orr.at[blk, pl.ds(t0, HALF_T), :])
        pltpu.sync_copy(sh, oh.at[blk, pl.ds(t0, HALF_T), :])

    return scatter_kernel


def _gru_body(ss_ref, rk_ref, b_ref, mz_ref, mr_ref, mh_ref, lens_ref, h0_ref,
              out_ref):
    s = lax.rsqrt(jnp.maximum(ss_ref[0, 0], 1e-12))
    rk0 = rk_ref[0, 0]
    rk1 = rk_ref[0, 1]
    rk2 = rk_ref[0, 2]
    b00 = b_ref[0, 0]
    b01 = b_ref[0, 1]
    b02 = b_ref[0, 2]
    b10 = b_ref[1, 0]
    b11 = b_ref[1, 1]
    b12 = b_ref[1, 2]
    h = h0_ref[...]
    lens = lens_ref[0] + lens_ref[1]
    for t in range(MAX_LEN):
        xz = mz_ref[:, t] * s + b00
        xr = mr_ref[:, t] * s + b01
        xh = mh_ref[:, t] * s + b02
        z = jax.nn.sigmoid(xz + h * rk0 + b10)
        r = jax.nn.sigmoid(xr + h * rk1 + b11)
        hh = jnp.tanh(xh + r * (h * rk2 + b12))
        h = jnp.where(t < lens, z * h + (1.0 - z) * hh, h)
    out_ref[...] = h


def _gru_scan(ss, rk, bias, mz, mr, mh, lens, h0):
    smem = pl.BlockSpec(memory_space=pltpu.SMEM)
    vmem = pl.BlockSpec(memory_space=pltpu.VMEM)
    return pl.pallas_call(
        _gru_body,
        in_specs=[smem, smem, smem, vmem, vmem, vmem, vmem, vmem],
        out_specs=pl.BlockSpec(memory_space=pltpu.VMEM),
        out_shape=jax.ShapeDtypeStruct((16, 128), jnp.float32),
    )(ss, rk, bias, mz, mr, mh, lens, h0)


def kernel(inputs, paths, index, sequences, features, flow_size,
           kernel, recurrent_kernel, bias):
    del features
    w_t = jnp.pad(kernel.T, ((0, 1), (0, 0)))
    p, ss = _project(inputs, w_t)

    total = paths.shape[0]
    padlen = total + (-total) % 16 + 32
    pad = padlen - total
    idx_p = jnp.pad(index, (0, pad), constant_values=B)
    seq_p = jnp.pad(sequences, (0, pad))
    path_p = jnp.pad(paths, (0, pad))

    mz, mr, mh, lens = _make_scatter(padlen, total)(
        p.reshape(-1), idx_p, seq_p, path_p)

    out = _gru_scan(ss, recurrent_kernel, bias, mz, mr, mh,
                    lens.reshape(2, 16, 128),
                    flow_size.reshape(16, 128))
    return out.reshape(NUM_QUESTS, NUM_PATHS)
